# Initial kernel scaffold; baseline (speedup 1.0000x reference)
import jax
import jax.numpy as jnp
from jax.experimental import pallas as pl


def kernel(edge_index, edge_values, emb):
    raise NotImplementedError("write your pallas kernel here")


# speedup vs baseline: 2.3109x; 2.3109x over previous
import functools

import jax
import jax.numpy as jnp
from jax import lax
from jax.experimental import pallas as pl
from jax.experimental.pallas import tpu as pltpu
from jax.experimental.pallas import tpu_sc as plsc

D = 64
NC = 2
NS = 16
L = 16
CHUNK = 128
N_LAYERS = 3


@functools.lru_cache(maxsize=None)
def _make_layer(n_nodes, e_pad):
    half = n_nodes // 2
    nch_tile = e_pad // (NS * CHUNK)
    span = nch_tile * CHUNK
    acc_rows = -(-(half + 1) // (NS * 8)) * (NS * 8)
    zrows_pt = acc_rows // NS
    trash = half
    out_pt = (half // NS) // 8 * 8
    out_main = out_pt * NS
    out_rem = half - out_main

    mesh = plsc.VectorSubcoreMesh(core_axis_name="c", subcore_axis_name="s")

    @functools.partial(
        pl.kernel,
        out_type=jax.ShapeDtypeStruct((n_nodes, D), jnp.float32),
        mesh=mesh,
        scratch_types=[
            pltpu.VMEM((CHUNK,), jnp.int32),
            pltpu.VMEM((CHUNK,), jnp.int32),
            pltpu.VMEM((CHUNK,), jnp.float32),
            pltpu.VMEM((CHUNK, D), jnp.float32),
            pltpu.VMEM_SHARED(
                (-(-(n_nodes // 2 + 1) // (NS * 8)) * (NS * 8), D),
                jnp.float32),
            pltpu.SemaphoreType.DMA,
        ],
        compiler_params=pltpu.CompilerParams(
            needs_layout_passes=False, use_tc_tiling_on_sc=False),
    )
    def layer(dst_hbm, src_hbm, val_hbm, table_hbm, out_hbm,
              idx_v, dst_v, val_v, rows_v, acc, sem):
        c = lax.axis_index("c")
        s = lax.axis_index("s")
        base_node = c * half

        def _zr(i, carry):
            for g in range(D // L):
                rows_v[i, pl.ds(g * L, L)] = jnp.zeros((L,), jnp.float32)
            return carry
        lax.fori_loop(0, CHUNK, _zr, 0)

        zbase = s * zrows_pt
        nfull = zrows_pt // CHUNK
        zrem = zrows_pt - nfull * CHUNK
        for k in range(nfull):
            pltpu.sync_copy(rows_v, acc.at[pl.ds(zbase + k * CHUNK, CHUNK)])
        if zrem:
            pltpu.sync_copy(rows_v.at[pl.ds(0, zrem)],
                            acc.at[pl.ds(zbase + nfull * CHUNK, zrem)])
        plsc.subcore_barrier()

        tile_base = s * span

        def _chunk(ch, carry):
            eb = tile_base + ch * CHUNK
            pltpu.sync_copy(src_hbm.at[pl.ds(eb, CHUNK)], idx_v)
            pltpu.sync_copy(dst_hbm.at[pl.ds(eb, CHUNK)], dst_v)
            pltpu.sync_copy(val_hbm.at[pl.ds(eb, CHUNK)], val_v)
            pltpu.async_copy(table_hbm.at[idx_v], rows_v, sem).wait()

            def _blk(b, carry2):
                off = b * L
                d = dst_v[pl.ds(off, L)] - base_node
                ok = (d >= 0) & (d < half)
                dst_v[pl.ds(off, L)] = jnp.where(ok, d, trash)
                for j in range(L):
                    e = off + j
                    ev = jnp.broadcast_to(e, (L,))
                    v = plsc.load_gather(val_v, [ev])
                    for g in range(D // L):
                        rows_v[e, pl.ds(g * L, L)] = (
                            rows_v[e, pl.ds(g * L, L)] * v)
                return carry2
            lax.fori_loop(0, CHUNK // L, _blk, 0)

            pltpu.sync_copy(rows_v, acc.at[dst_v], add=True)
            return carry
        lax.fori_loop(0, nch_tile, _chunk, 0)

        plsc.subcore_barrier()

        ob = s * out_pt
        pltpu.sync_copy(acc.at[pl.ds(ob, out_pt)],
                        out_hbm.at[pl.ds(base_node + ob, out_pt)])
        if out_rem:
            @pl.when(s == 0)
            def _rem():
                pltpu.sync_copy(
                    acc.at[pl.ds(out_main, out_rem)],
                    out_hbm.at[pl.ds(base_node + out_main, out_rem)])

    return layer


@functools.lru_cache(maxsize=None)
def _make_mean(n_nodes):
    blk = 1000
    grid = n_nodes // blk

    def body(a, b, c, d, o):
        o[...] = (a[...] + b[...] + c[...] + d[...]) * 0.25

    return pl.pallas_call(
        body,
        grid=(grid,),
        in_specs=[pl.BlockSpec((blk, D), lambda i: (i, 0))] * 4,
        out_specs=pl.BlockSpec((blk, D), lambda i: (i, 0)),
        out_shape=jax.ShapeDtypeStruct((n_nodes, D), jnp.float32),
    )


@jax.jit
def _impl(edge_index, edge_values, emb):
    n_nodes = emb.shape[0]
    n_edges = edge_values.shape[0]
    grp = NS * CHUNK
    e_pad = -(-n_edges // grp) * grp
    pad = e_pad - n_edges
    row = edge_index[0].astype(jnp.int32)
    col = edge_index[1].astype(jnp.int32)
    val = edge_values
    if pad:
        zi = jnp.zeros((pad,), jnp.int32)
        row = jnp.concatenate([row, zi])
        col = jnp.concatenate([col, zi])
        val = jnp.concatenate([val, jnp.zeros((pad,), val.dtype)])

    layer = _make_layer(n_nodes, e_pad)
    embs = [emb]
    for _ in range(N_LAYERS):
        embs.append(layer(row, col, val, embs[-1]))
    mean = _make_mean(n_nodes)(*embs)
    half = n_nodes // 2
    return mean[:half], mean[half:]


def kernel(edge_index, edge_values, emb):
    return _impl(edge_index, edge_values, emb)


# speedup vs baseline: 4.2472x; 1.8379x over previous
import functools

import jax
import jax.numpy as jnp
from jax import lax
from jax.experimental import pallas as pl
from jax.experimental.pallas import tpu as pltpu
from jax.experimental.pallas import tpu_sc as plsc

D = 64
NC = 2
NS = 16
L = 16
CHUNK = 96
CPS = 2
SUP = CHUNK * CPS
N_LAYERS = 3


@functools.lru_cache(maxsize=None)
def _make_layer(n_nodes, e_pad):
    half = n_nodes // 2
    span = e_pad // NS
    nsup = span // SUP
    npairs = nsup // 2
    acc_rows = -(-(half + 1) // (NS * 8)) * (NS * 8)
    zrows_pt = acc_rows // NS
    trash = half
    out_pt = (half // NS) // 8 * 8
    out_main = out_pt * NS
    out_rem = half - out_main

    mesh = plsc.VectorSubcoreMesh(core_axis_name="c", subcore_axis_name="s")

    @functools.partial(
        pl.kernel,
        out_type=jax.ShapeDtypeStruct((n_nodes, D), jnp.float32),
        mesh=mesh,
        scratch_types=[
            pltpu.VMEM((SUP,), jnp.int32),
            pltpu.VMEM((SUP,), jnp.int32),
            pltpu.VMEM((SUP,), jnp.float32),
            pltpu.VMEM((CPS, CHUNK), jnp.int32),
            pltpu.VMEM((SUP, D), jnp.float32),
            pltpu.VMEM((SUP,), jnp.int32),
            pltpu.VMEM((SUP,), jnp.int32),
            pltpu.VMEM((SUP,), jnp.float32),
            pltpu.VMEM((CPS, CHUNK), jnp.int32),
            pltpu.VMEM((SUP, D), jnp.float32),
            pltpu.VMEM_SHARED(
                (-(-(n_nodes // 2 + 1) // (NS * 8)) * (NS * 8), D),
                jnp.float32),
            pltpu.SemaphoreType.DMA,
            pltpu.SemaphoreType.DMA,
            pltpu.SemaphoreType.DMA,
            pltpu.SemaphoreType.DMA,
            pltpu.SemaphoreType.DMA,
            pltpu.SemaphoreType.DMA,
        ],
        compiler_params=pltpu.CompilerParams(
            needs_layout_passes=False, use_tc_tiling_on_sc=False),
    )
    def layer(dst_hbm, src_hbm, val_hbm, table_hbm, out_hbm,
              srcA, dstA, valA, dstlocA, rowsA,
              srcB, dstB, valB, dstlocB, rowsB,
              acc, msemA, msemB, gsemA, gsemB, ssemA, ssemB):
        c = lax.axis_index("c")
        s = lax.axis_index("s")
        base_node = c * half
        tile_base = s * span

        bufs = {
            0: (srcA, dstA, valA, dstlocA, rowsA, msemA, gsemA, ssemA),
            1: (srcB, dstB, valB, dstlocB, rowsB, msemB, gsemB, ssemB),
        }

        def fire_meta(p, sup):
            src_v, dst_v, val_v = bufs[p][0], bufs[p][1], bufs[p][2]
            msem = bufs[p][5]
            mb = tile_base + sup * SUP
            pltpu.async_copy(src_hbm.at[pl.ds(mb, SUP)], src_v, msem)
            pltpu.async_copy(dst_hbm.at[pl.ds(mb, SUP)], dst_v, msem)
            pltpu.async_copy(val_hbm.at[pl.ds(mb, SUP)], val_v, msem)

        def wait_meta(p):
            src_v, dst_v, val_v = bufs[p][0], bufs[p][1], bufs[p][2]
            msem = bufs[p][5]
            pltpu.make_async_copy(src_hbm.at[pl.ds(0, SUP)], src_v,
                                  msem).wait()
            pltpu.make_async_copy(dst_hbm.at[pl.ds(0, SUP)], dst_v,
                                  msem).wait()
            pltpu.make_async_copy(val_hbm.at[pl.ds(0, SUP)], val_v,
                                  msem).wait()

        def fire_gathers(p):
            src_v, rows_v, gsem = bufs[p][0], bufs[p][4], bufs[p][6]
            for k in range(CPS):
                pltpu.async_copy(
                    table_hbm.at[src_v.at[pl.ds(k * CHUNK, CHUNK)]],
                    rows_v.at[pl.ds(k * CHUNK, CHUNK)], gsem)

        def wait_gathers(p):
            rows_v, gsem = bufs[p][4], bufs[p][6]
            pltpu.make_async_copy(table_hbm.at[pl.ds(0, SUP)], rows_v,
                                  gsem).wait()

        def fire_scatters(p):
            dstloc_v, rows_v, ssem = bufs[p][3], bufs[p][4], bufs[p][7]
            for k in range(CPS):
                pltpu.async_copy(
                    rows_v.at[pl.ds(k * CHUNK, CHUNK)],
                    acc.at[dstloc_v.at[k]], ssem, add=True)

        def wait_scatters(p):
            rows_v, ssem = bufs[p][4], bufs[p][7]
            pltpu.make_async_copy(table_hbm.at[pl.ds(0, SUP)], rows_v,
                                  ssem).wait()

        def compute(p):
            dst_v, val_v, dstloc_v, rows_v = (
                bufs[p][1], bufs[p][2], bufs[p][3], bufs[p][4])

            def blk(b, carry):
                off = b * L
                d = dst_v[pl.ds(off, L)] - base_node
                ok = (d >= 0) & (d < half)
                dstloc_v[b // (CHUNK // L),
                         pl.ds((b % (CHUNK // L)) * L, L)] = (
                    jnp.where(ok, d, trash))
                for j in range(L):
                    e = off + j
                    v = plsc.load_gather(val_v, [jnp.broadcast_to(e, (L,))])
                    for g in range(D // L):
                        rows_v[e, pl.ds(g * L, L)] = (
                            rows_v[e, pl.ds(g * L, L)] * v)
                return carry
            lax.fori_loop(0, SUP // L, blk, 0)

        def _zr(i, carry):
            for g in range(D // L):
                rowsA[i, pl.ds(g * L, L)] = jnp.zeros((L,), jnp.float32)
            return carry
        lax.fori_loop(0, SUP, _zr, 0)

        zbase = s * zrows_pt
        nfull = zrows_pt // SUP
        zrem = zrows_pt - nfull * SUP
        for k in range(nfull):
            pltpu.sync_copy(rowsA, acc.at[pl.ds(zbase + k * SUP, SUP)])
        if zrem:
            pltpu.sync_copy(rowsA.at[pl.ds(0, zrem)],
                            acc.at[pl.ds(zbase + nfull * SUP, zrem)])
        plsc.subcore_barrier()

        fire_meta(0, 0)
        wait_meta(0)
        fire_gathers(0)
        fire_meta(1, 1)

        def pair(i, carry):
            interior = i < npairs - 1
            wait_meta(1)

            @pl.when(i > 0)
            def _():
                wait_scatters(1)
            fire_gathers(1)
            wait_gathers(0)
            compute(0)
            fire_scatters(0)

            @pl.when(interior)
            def _():
                fire_meta(0, 2 * i + 2)
            wait_gathers(1)
            compute(1)
            fire_scatters(1)

            @pl.when(interior)
            def _():
                wait_meta(0)
                wait_scatters(0)
                fire_gathers(0)
                fire_meta(1, 2 * i + 3)
            return carry
        lax.fori_loop(0, npairs, pair, 0)
        wait_scatters(0)
        wait_scatters(1)

        plsc.subcore_barrier()

        ob = s * out_pt
        pltpu.sync_copy(acc.at[pl.ds(ob, out_pt)],
                        out_hbm.at[pl.ds(base_node + ob, out_pt)])
        if out_rem:
            @pl.when(s == 0)
            def _rem():
                pltpu.sync_copy(
                    acc.at[pl.ds(out_main, out_rem)],
                    out_hbm.at[pl.ds(base_node + out_main, out_rem)])

    return layer


@functools.lru_cache(maxsize=None)
def _make_mean(n_nodes):
    blk = 1000
    grid = n_nodes // blk

    def body(a, b, c, d, o):
        o[...] = (a[...] + b[...] + c[...] + d[...]) * 0.25

    return pl.pallas_call(
        body,
        grid=(grid,),
        in_specs=[pl.BlockSpec((blk, D), lambda i: (i, 0))] * 4,
        out_specs=pl.BlockSpec((blk, D), lambda i: (i, 0)),
        out_shape=jax.ShapeDtypeStruct((n_nodes, D), jnp.float32),
    )


@jax.jit
def _impl(edge_index, edge_values, emb):
    n_nodes = emb.shape[0]
    n_edges = edge_values.shape[0]
    grp = NS * SUP * 2
    e_pad = -(-n_edges // grp) * grp
    pad = e_pad - n_edges
    row = edge_index[0].astype(jnp.int32)
    col = edge_index[1].astype(jnp.int32)
    val = edge_values
    if pad:
        zi = jnp.zeros((pad,), jnp.int32)
        row = jnp.concatenate([row, zi])
        col = jnp.concatenate([col, zi])
        val = jnp.concatenate([val, jnp.zeros((pad,), val.dtype)])

    layer = _make_layer(n_nodes, e_pad)
    embs = [emb]
    for _ in range(N_LAYERS):
        embs.append(layer(row, col, val, embs[-1]))
    mean = _make_mean(n_nodes)(*embs)
    half = n_nodes // 2
    return mean[:half], mean[half:]


def kernel(edge_index, edge_values, emb):
    return _impl(edge_index, edge_values, emb)


# speedup vs baseline: 4.5867x; 1.0799x over previous
import functools

import jax
import jax.numpy as jnp
from jax import lax
from jax.experimental import pallas as pl
from jax.experimental.pallas import tpu as pltpu
from jax.experimental.pallas import tpu_sc as plsc

D = 64
NC = 2
NS = 16
NW = NC * NS
L = 16
CHUNK = 96
CPS = 2
SUP = CHUNK * CPS
PAIR = 2 * SUP
N_LAYERS = 3

MS = 256
W = 2304
WSTRIDE = W + 400
SCAP = WSTRIDE + W + 400


def _rcap(e_pad):
    span32 = e_pad // NW
    return -(-span32 // W) * W


@functools.lru_cache(maxsize=None)
def _make_partition(n_nodes, e_pad):
    half = n_nodes // 2
    span32 = e_pad // NW
    nms = span32 // MS
    nmp = nms // 2
    rcap = _rcap(e_pad)

    mesh = plsc.VectorSubcoreMesh(core_axis_name="c", subcore_axis_name="s")

    @functools.partial(
        pl.kernel,
        out_type=(
            jax.ShapeDtypeStruct((NC * NW * rcap,), jnp.int32),
            jax.ShapeDtypeStruct((NC * NW * rcap,), jnp.int32),
            jax.ShapeDtypeStruct((NC * NW * rcap,), jnp.float32),
            jax.ShapeDtypeStruct((NC * NW * L,), jnp.int32),
        ),
        mesh=mesh,
        scratch_types=[
            pltpu.VMEM((MS,), jnp.int32),
            pltpu.VMEM((MS,), jnp.int32),
            pltpu.VMEM((MS,), jnp.float32),
            pltpu.VMEM((MS,), jnp.int32),
            pltpu.VMEM((MS,), jnp.int32),
            pltpu.VMEM((MS,), jnp.float32),
            pltpu.VMEM((SCAP,), jnp.int32),
            pltpu.VMEM((SCAP,), jnp.int32),
            pltpu.VMEM((SCAP,), jnp.float32),
            pltpu.VMEM((SCAP,), jnp.int32),
            pltpu.VMEM((SCAP,), jnp.int32),
            pltpu.VMEM((SCAP,), jnp.float32),
            pltpu.VMEM((L,), jnp.int32),
            pltpu.SemaphoreType.DMA,
            pltpu.SemaphoreType.DMA,
            pltpu.SemaphoreType.DMA,
            pltpu.SemaphoreType.DMA,
        ],
        compiler_params=pltpu.CompilerParams(
            needs_layout_passes=False, use_tc_tiling_on_sc=False),
    )
    def part(dst_hbm, src_hbm, val_hbm,
             srcP, dstP, valP, cntP,
             srcMA, dstMA, valMA, srcMB, dstMB, valMB,
             stg_s0, stg_d0, stg_v0, stg_s1, stg_d1, stg_v1,
             cnt_stg, msemA, msemB, fsem0, fsem1):
        c = lax.axis_index("c")
        s = lax.axis_index("s")
        w = c * NS + s
        wbase = w * span32

        metas = {0: (srcMA, dstMA, valMA, msemA),
                 1: (srcMB, dstMB, valMB, msemB)}
        stgs = {0: (stg_s0, stg_d0, stg_v0, fsem0),
                1: (stg_s1, stg_d1, stg_v1, fsem1)}

        def fire_meta(p, m):
            sm, dm, vm, msem = metas[p]
            mb = wbase + m * MS
            pltpu.async_copy(src_hbm.at[pl.ds(mb, MS)], sm, msem)
            pltpu.async_copy(dst_hbm.at[pl.ds(mb, MS)], dm, msem)
            pltpu.async_copy(val_hbm.at[pl.ds(mb, MS)], vm, msem)

        def wait_meta(p):
            sm, dm, vm, msem = metas[p]
            pltpu.make_async_copy(src_hbm.at[pl.ds(0, MS)], sm, msem).wait()
            pltpu.make_async_copy(dst_hbm.at[pl.ds(0, MS)], dm, msem).wait()
            pltpu.make_async_copy(val_hbm.at[pl.ds(0, MS)], vm, msem).wait()

        iota = lax.iota(jnp.int32, L)

        def flush_wait(cc):
            st, dt, vt, fsem = stgs[cc]
            pltpu.make_async_copy(src_hbm.at[pl.ds(0, W)], st.at[pl.ds(0, W)],
                                  fsem).wait()
            pltpu.make_async_copy(src_hbm.at[pl.ds(0, W)], dt.at[pl.ds(0, W)],
                                  fsem).wait()
            pltpu.make_async_copy(val_hbm.at[pl.ds(0, W)], vt.at[pl.ds(0, W)],
                                  fsem).wait()

        def flush_fire(cc, wb, nf):
            st, dt, vt, fsem = stgs[cc]
            ob = ((cc * NW + w) * (rcap // W) + nf) * W
            pltpu.async_copy(st.at[pl.ds(wb, W)], srcP.at[pl.ds(ob, W)], fsem)
            pltpu.async_copy(dt.at[pl.ds(wb, W)], dstP.at[pl.ds(ob, W)], fsem)
            pltpu.async_copy(vt.at[pl.ds(wb, W)], valP.at[pl.ds(ob, W)], fsem)

        def append(cc, sv, dv, vv, mask, pos, out, nf):
            st, dt, vt, _ = stgs[cc]
            wb = (nf % 2) * WSTRIDE
            at = wb + pos
            plsc.store_compressed(st.at[pl.ds(at, L)], sv, mask=mask)
            plsc.store_compressed(dt.at[pl.ds(at, L)], dv, mask=mask)
            plsc.store_compressed(vt.at[pl.ds(at, L)], vv, mask=mask)
            n = jnp.max(plsc.all_reduce_population_count(mask))
            pos = pos + n
            cross = pos >= W

            def do_flush():
                sp_s = st[pl.ds(wb + W, L)]
                sp_d = dt[pl.ds(wb + W, L)]
                sp_v = vt[pl.ds(wb + W, L)]

                @pl.when(nf > 0)
                def _():
                    flush_wait(cc)
                flush_fire(cc, wb, nf)
                nwb = ((nf + 1) % 2) * WSTRIDE
                rem = pos - W
                m = iota < rem
                plsc.store_compressed(st.at[pl.ds(nwb, L)], sp_s, mask=m)
                plsc.store_compressed(dt.at[pl.ds(nwb, L)], sp_d, mask=m)
                plsc.store_compressed(vt.at[pl.ds(nwb, L)], sp_v, mask=m)

            @pl.when(cross)
            def _():
                do_flush()
            pos = jnp.where(cross, pos - W, pos)
            out = jnp.where(cross, out + W, out)
            nf = jnp.where(cross, nf + 1, nf)
            return pos, out, nf

        def do_msup(p, carry):
            sm, dm, vm, _ = metas[p]

            def grp(g, cr):
                pos0, out0, nf0, pos1, out1, nf1 = cr
                off = g * L
                sv = sm[pl.ds(off, L)]
                dv = dm[pl.ds(off, L)]
                vv = vm[pl.ds(off, L)]
                own0 = dv < half
                dloc = jnp.where(own0, dv, dv - half)
                pos0, out0, nf0 = append(0, sv, dloc, vv, own0,
                                         pos0, out0, nf0)
                pos1, out1, nf1 = append(1, sv, dloc, vv,
                                         jnp.logical_not(own0),
                                         pos1, out1, nf1)
                return (pos0, out0, nf0, pos1, out1, nf1)
            return lax.fori_loop(0, MS // L, grp, carry)

        fire_meta(0, 0)
        fire_meta(1, 1)

        def mpair(i, carry):
            wait_meta(0)
            carry = do_msup(0, carry)

            @pl.when(i < nmp - 1)
            def _():
                fire_meta(0, 2 * i + 2)
            wait_meta(1)
            carry = do_msup(1, carry)

            @pl.when(i < nmp - 1)
            def _():
                fire_meta(1, 2 * i + 3)
            return carry

        z = jnp.int32(0)
        carry = lax.fori_loop(0, nmp, mpair, (z, z, z, z, z, z))
        pos0, out0, nf0, pos1, out1, nf1 = carry

        def finish(cc, pos, out, nf):
            st, dt, vt, _ = stgs[cc]
            wb = (nf % 2) * WSTRIDE
            zs = jnp.zeros((L,), jnp.int32)
            zt = jnp.full((L,), half, jnp.int32)
            zv = jnp.zeros((L,), jnp.float32)
            for k in range(PAIR // L):
                at = wb + pos + k * L
                st[pl.ds(at, L)] = zs
                dt[pl.ds(at, L)] = zt
                vt[pl.ds(at, L)] = zv

            @pl.when(nf > 0)
            def _():
                flush_wait(cc)
            flush_fire(cc, wb, nf)
            flush_wait(cc)
            cnt = out + ((pos + PAIR - 1) // PAIR) * PAIR
            return cnt

        cnt0 = finish(0, pos0, out0, nf0)
        cnt1 = finish(1, pos1, out1, nf1)

        for cc, cnt in ((0, cnt0), (1, cnt1)):
            cnt_stg[pl.ds(0, L)] = jnp.broadcast_to(cnt, (L,))
            pltpu.sync_copy(cnt_stg, cntP.at[pl.ds((cc * NW + w) * L, L)])

    return part


@functools.lru_cache(maxsize=None)
def _make_layer(n_nodes, e_pad):
    half = n_nodes // 2
    rcap = _rcap(e_pad)
    acc_rows = -(-(half + 1) // (NS * 8)) * (NS * 8)
    zrows_pt = acc_rows // NS
    trash = half
    out_pt = (half // NS) // 8 * 8
    out_main = out_pt * NS
    out_rem = half - out_main

    mesh = plsc.VectorSubcoreMesh(core_axis_name="c", subcore_axis_name="s")

    @functools.partial(
        pl.kernel,
        out_type=jax.ShapeDtypeStruct((n_nodes, D), jnp.float32),
        mesh=mesh,
        scratch_types=[
            pltpu.VMEM((SUP,), jnp.int32),
            pltpu.VMEM((SUP,), jnp.int32),
            pltpu.VMEM((SUP,), jnp.float32),
            pltpu.VMEM((CPS, CHUNK), jnp.int32),
            pltpu.VMEM((SUP, D), jnp.float32),
            pltpu.VMEM((SUP,), jnp.int32),
            pltpu.VMEM((SUP,), jnp.int32),
            pltpu.VMEM((SUP,), jnp.float32),
            pltpu.VMEM((CPS, CHUNK), jnp.int32),
            pltpu.VMEM((SUP, D), jnp.float32),
            pltpu.VMEM((L,), jnp.int32),
            pltpu.VMEM_SHARED(
                (-(-(n_nodes // 2 + 1) // (NS * 8)) * (NS * 8), D),
                jnp.float32),
            pltpu.SemaphoreType.DMA,
            pltpu.SemaphoreType.DMA,
            pltpu.SemaphoreType.DMA,
            pltpu.SemaphoreType.DMA,
            pltpu.SemaphoreType.DMA,
            pltpu.SemaphoreType.DMA,
        ],
        compiler_params=pltpu.CompilerParams(
            needs_layout_passes=False, use_tc_tiling_on_sc=False),
    )
    def layer(srcP, dstP, valP, cntP, table_hbm, out_hbm,
              srcA, dstA, valA, dstlocA, rowsA,
              srcB, dstB, valB, dstlocB, rowsB,
              cnt_stg, acc, msemA, msemB, gsemA, gsemB, ssemA, ssemB):
        c = lax.axis_index("c")
        s = lax.axis_index("s")
        base_node = c * half

        bufs = {
            0: (srcA, dstA, valA, dstlocA, rowsA, msemA, gsemA, ssemA),
            1: (srcB, dstB, valB, dstlocB, rowsB, msemB, gsemB, ssemB),
        }

        def fire_meta(p, rb, sup):
            src_v, dst_v, val_v = bufs[p][0], bufs[p][1], bufs[p][2]
            msem = bufs[p][5]
            mb = (rb + sup) * SUP
            pltpu.async_copy(srcP.at[pl.ds(mb, SUP)], src_v, msem)
            pltpu.async_copy(dstP.at[pl.ds(mb, SUP)], dst_v, msem)
            pltpu.async_copy(valP.at[pl.ds(mb, SUP)], val_v, msem)

        def wait_meta(p):
            src_v, dst_v, val_v = bufs[p][0], bufs[p][1], bufs[p][2]
            msem = bufs[p][5]
            pltpu.make_async_copy(srcP.at[pl.ds(0, SUP)], src_v, msem).wait()
            pltpu.make_async_copy(dstP.at[pl.ds(0, SUP)], dst_v, msem).wait()
            pltpu.make_async_copy(valP.at[pl.ds(0, SUP)], val_v, msem).wait()

        def fire_gathers(p):
            src_v, rows_v, gsem = bufs[p][0], bufs[p][4], bufs[p][6]
            for k in range(CPS):
                pltpu.async_copy(
                    table_hbm.at[src_v.at[pl.ds(k * CHUNK, CHUNK)]],
                    rows_v.at[pl.ds(k * CHUNK, CHUNK)], gsem)

        def wait_gathers(p):
            rows_v, gsem = bufs[p][4], bufs[p][6]
            pltpu.make_async_copy(table_hbm.at[pl.ds(0, SUP)], rows_v,
                                  gsem).wait()

        def fire_scatters(p):
            dstloc_v, rows_v, ssem = bufs[p][3], bufs[p][4], bufs[p][7]
            for k in range(CPS):
                pltpu.async_copy(
                    rows_v.at[pl.ds(k * CHUNK, CHUNK)],
                    acc.at[dstloc_v.at[k]], ssem, add=True)

        def wait_scatters(p):
            rows_v, ssem = bufs[p][4], bufs[p][7]
            pltpu.make_async_copy(table_hbm.at[pl.ds(0, SUP)], rows_v,
                                  ssem).wait()

        def compute(p):
            dst_v, val_v, dstloc_v, rows_v = (
                bufs[p][1], bufs[p][2], bufs[p][3], bufs[p][4])

            def blk(b, carry):
                off = b * L
                dstloc_v[b // (CHUNK // L),
                         pl.ds((b % (CHUNK // L)) * L, L)] = (
                    dst_v[pl.ds(off, L)])
                for j in range(L):
                    e = off + j
                    v = plsc.load_gather(val_v, [jnp.broadcast_to(e, (L,))])
                    for g in range(D // L):
                        rows_v[e, pl.ds(g * L, L)] = (
                            rows_v[e, pl.ds(g * L, L)] * v)
                return carry
            lax.fori_loop(0, SUP // L, blk, 0)

        def _zr(i, carry):
            for g in range(D // L):
                rowsA[i, pl.ds(g * L, L)] = jnp.zeros((L,), jnp.float32)
            return carry
        lax.fori_loop(0, SUP, _zr, 0)

        zbase = s * zrows_pt
        nfull = zrows_pt // SUP
        zrem = zrows_pt - nfull * SUP
        for k in range(nfull):
            pltpu.sync_copy(rowsA, acc.at[pl.ds(zbase + k * SUP, SUP)])
        if zrem:
            pltpu.sync_copy(rowsA.at[pl.ds(0, zrem)],
                            acc.at[pl.ds(zbase + nfull * SUP, zrem)])
        plsc.subcore_barrier()

        def do_region(r):
            rb = (c * NW + r) * (rcap // SUP)
            pltpu.sync_copy(cntP.at[pl.ds((c * NW + r) * L, L)], cnt_stg)
            cnt = jnp.max(cnt_stg[pl.ds(0, L)])
            npairs = cnt // PAIR

            @pl.when(npairs > 0)
            def _():
                fire_meta(0, rb, 0)
                wait_meta(0)
                fire_gathers(0)
                fire_meta(1, rb, 1)

                def pair(i, carry):
                    interior = i < npairs - 1
                    wait_meta(1)

                    @pl.when(i > 0)
                    def _():
                        wait_scatters(1)
                    fire_gathers(1)
                    wait_gathers(0)
                    compute(0)
                    fire_scatters(0)

                    @pl.when(interior)
                    def _():
                        fire_meta(0, rb, 2 * i + 2)
                    wait_gathers(1)
                    compute(1)
                    fire_scatters(1)

                    @pl.when(interior)
                    def _():
                        wait_meta(0)
                        wait_scatters(0)
                        fire_gathers(0)
                        fire_meta(1, rb, 2 * i + 3)
                    return carry
                lax.fori_loop(0, npairs, pair, 0)
                wait_scatters(0)
                wait_scatters(1)

        do_region(2 * s)
        do_region(2 * s + 1)

        plsc.subcore_barrier()

        ob = s * out_pt
        pltpu.sync_copy(acc.at[pl.ds(ob, out_pt)],
                        out_hbm.at[pl.ds(base_node + ob, out_pt)])
        if out_rem:
            @pl.when(s == 0)
            def _rem():
                pltpu.sync_copy(
                    acc.at[pl.ds(out_main, out_rem)],
                    out_hbm.at[pl.ds(base_node + out_main, out_rem)])

    return layer


@functools.lru_cache(maxsize=None)
def _make_mean(n_nodes):
    blk = 1000
    grid = n_nodes // blk

    def body(a, b, c, d, o):
        o[...] = (a[...] + b[...] + c[...] + d[...]) * 0.25

    return pl.pallas_call(
        body,
        grid=(grid,),
        in_specs=[pl.BlockSpec((blk, D), lambda i: (i, 0))] * 4,
        out_specs=pl.BlockSpec((blk, D), lambda i: (i, 0)),
        out_shape=jax.ShapeDtypeStruct((n_nodes, D), jnp.float32),
    )


@jax.jit
def _impl(edge_index, edge_values, emb):
    n_nodes = emb.shape[0]
    n_edges = edge_values.shape[0]
    grp = NW * MS * 2
    e_pad = -(-n_edges // grp) * grp
    pad = e_pad - n_edges
    row = edge_index[0].astype(jnp.int32)
    col = edge_index[1].astype(jnp.int32)
    val = edge_values
    if pad:
        zi = jnp.zeros((pad,), jnp.int32)
        row = jnp.concatenate([row, zi])
        col = jnp.concatenate([col, zi])
        val = jnp.concatenate([val, jnp.zeros((pad,), val.dtype)])

    srcP, dstP, valP, cntP = _make_partition(n_nodes, e_pad)(row, col, val)
    layer = _make_layer(n_nodes, e_pad)
    embs = [emb]
    for _ in range(N_LAYERS):
        embs.append(layer(srcP, dstP, valP, cntP, embs[-1]))
    mean = _make_mean(n_nodes)(*embs)
    half = n_nodes // 2
    return mean[:half], mean[half:]


def kernel(edge_index, edge_values, emb):
    return _impl(edge_index, edge_values, emb)


# speedup vs baseline: 6.5378x; 1.4254x over previous
import functools

import jax
import jax.numpy as jnp
from jax import lax
from jax.experimental import pallas as pl
from jax.experimental.pallas import tpu as pltpu
from jax.experimental.pallas import tpu_sc as plsc

D = 64
NC = 2
NS = 16
NW = NC * NS
L = 16
CHUNK = 96
CPS = 2
SUP = CHUNK * CPS
PAIR = 2 * SUP
N_LAYERS = 3

MS = 256
W = 2304
WSTRIDE = W + PAIR + 32
SCAP = WSTRIDE + W + PAIR + 32


def _rcap(e_pad):
    span32 = e_pad // NW
    return -(-span32 // W) * W


@functools.lru_cache(maxsize=None)
def _make_partition(n_nodes, e_pad):
    half = n_nodes // 2
    span32 = e_pad // NW
    nms = span32 // MS
    nmp = nms // 2
    rcap = _rcap(e_pad)

    mesh = plsc.VectorSubcoreMesh(core_axis_name="c", subcore_axis_name="s")

    @functools.partial(
        pl.kernel,
        out_type=(
            jax.ShapeDtypeStruct((NC * NW * rcap,), jnp.int32),
            jax.ShapeDtypeStruct((NC * NW * rcap,), jnp.int32),
            jax.ShapeDtypeStruct((NC * NW * rcap,), jnp.float32),
            jax.ShapeDtypeStruct((NC * NW * L,), jnp.int32),
        ),
        mesh=mesh,
        scratch_types=[
            pltpu.VMEM((MS,), jnp.int32),
            pltpu.VMEM((MS,), jnp.int32),
            pltpu.VMEM((MS,), jnp.float32),
            pltpu.VMEM((MS,), jnp.int32),
            pltpu.VMEM((MS,), jnp.int32),
            pltpu.VMEM((MS,), jnp.float32),
            pltpu.VMEM((SCAP,), jnp.int32),
            pltpu.VMEM((SCAP,), jnp.int32),
            pltpu.VMEM((SCAP,), jnp.float32),
            pltpu.VMEM((SCAP,), jnp.int32),
            pltpu.VMEM((SCAP,), jnp.int32),
            pltpu.VMEM((SCAP,), jnp.float32),
            pltpu.VMEM((L,), jnp.int32),
            pltpu.SemaphoreType.DMA,
            pltpu.SemaphoreType.DMA,
            pltpu.SemaphoreType.DMA,
            pltpu.SemaphoreType.DMA,
        ],
        compiler_params=pltpu.CompilerParams(
            needs_layout_passes=False, use_tc_tiling_on_sc=False),
    )
    def part(dst_hbm, src_hbm, val_hbm,
             srcP, dstP, valP, cntP,
             srcMA, dstMA, valMA, srcMB, dstMB, valMB,
             stg_s0, stg_d0, stg_v0, stg_s1, stg_d1, stg_v1,
             cnt_stg, msemA, msemB, fsem0, fsem1):
        c = lax.axis_index("c")
        s = lax.axis_index("s")
        w = c * NS + s
        wbase = w * span32

        metas = {0: (srcMA, dstMA, valMA, msemA),
                 1: (srcMB, dstMB, valMB, msemB)}
        stgs = {0: (stg_s0, stg_d0, stg_v0, fsem0),
                1: (stg_s1, stg_d1, stg_v1, fsem1)}

        def fire_meta(p, m):
            sm, dm, vm, msem = metas[p]
            mb = wbase + m * MS
            pltpu.async_copy(src_hbm.at[pl.ds(mb, MS)], sm, msem)
            pltpu.async_copy(dst_hbm.at[pl.ds(mb, MS)], dm, msem)
            pltpu.async_copy(val_hbm.at[pl.ds(mb, MS)], vm, msem)

        def wait_meta(p):
            sm, dm, vm, msem = metas[p]
            pltpu.make_async_copy(src_hbm.at[pl.ds(0, MS)], sm, msem).wait()
            pltpu.make_async_copy(dst_hbm.at[pl.ds(0, MS)], dm, msem).wait()
            pltpu.make_async_copy(val_hbm.at[pl.ds(0, MS)], vm, msem).wait()

        iota = lax.iota(jnp.int32, L)

        def flush_wait(cc):
            st, dt, vt, fsem = stgs[cc]
            pltpu.make_async_copy(src_hbm.at[pl.ds(0, W)], st.at[pl.ds(0, W)],
                                  fsem).wait()
            pltpu.make_async_copy(src_hbm.at[pl.ds(0, W)], dt.at[pl.ds(0, W)],
                                  fsem).wait()
            pltpu.make_async_copy(val_hbm.at[pl.ds(0, W)], vt.at[pl.ds(0, W)],
                                  fsem).wait()

        def flush_fire(cc, wb, nf):
            st, dt, vt, fsem = stgs[cc]
            ob = ((cc * NW + w) * (rcap // W) + nf) * W
            pltpu.async_copy(st.at[pl.ds(wb, W)], srcP.at[pl.ds(ob, W)], fsem)
            pltpu.async_copy(dt.at[pl.ds(wb, W)], dstP.at[pl.ds(ob, W)], fsem)
            pltpu.async_copy(vt.at[pl.ds(wb, W)], valP.at[pl.ds(ob, W)], fsem)

        def append(cc, sv, dv, vv, mask, pos, out, nf):
            st, dt, vt, _ = stgs[cc]
            wb = (nf % 2) * WSTRIDE
            at = wb + pos
            plsc.store_compressed(st.at[pl.ds(at, L)], sv, mask=mask)
            plsc.store_compressed(dt.at[pl.ds(at, L)], dv, mask=mask)
            plsc.store_compressed(vt.at[pl.ds(at, L)], vv, mask=mask)
            n = jnp.max(plsc.all_reduce_population_count(mask))
            pos = pos + n
            cross = pos >= W

            def do_flush():
                sp_s = st[pl.ds(wb + W, L)]
                sp_d = dt[pl.ds(wb + W, L)]
                sp_v = vt[pl.ds(wb + W, L)]

                @pl.when(nf > 0)
                def _():
                    flush_wait(cc)
                flush_fire(cc, wb, nf)
                nwb = ((nf + 1) % 2) * WSTRIDE
                rem = pos - W
                m = iota < rem
                plsc.store_compressed(st.at[pl.ds(nwb, L)], sp_s, mask=m)
                plsc.store_compressed(dt.at[pl.ds(nwb, L)], sp_d, mask=m)
                plsc.store_compressed(vt.at[pl.ds(nwb, L)], sp_v, mask=m)

            @pl.when(cross)
            def _():
                do_flush()
            pos = jnp.where(cross, pos - W, pos)
            out = jnp.where(cross, out + W, out)
            nf = jnp.where(cross, nf + 1, nf)
            return pos, out, nf

        def do_msup(p, carry):
            sm, dm, vm, _ = metas[p]

            def grp(g, cr):
                pos0, out0, nf0, pos1, out1, nf1 = cr
                off = g * L
                sv = sm[pl.ds(off, L)]
                dv = dm[pl.ds(off, L)]
                vv = vm[pl.ds(off, L)]
                own0 = dv < half
                dloc = jnp.where(own0, dv, dv - half)
                pos0, out0, nf0 = append(0, sv, dloc, vv, own0,
                                         pos0, out0, nf0)
                pos1, out1, nf1 = append(1, sv, dloc, vv,
                                         jnp.logical_not(own0),
                                         pos1, out1, nf1)
                return (pos0, out0, nf0, pos1, out1, nf1)
            return lax.fori_loop(0, MS // L, grp, carry)

        fire_meta(0, 0)
        fire_meta(1, 1)

        def mpair(i, carry):
            wait_meta(0)
            carry = do_msup(0, carry)

            @pl.when(i < nmp - 1)
            def _():
                fire_meta(0, 2 * i + 2)
            wait_meta(1)
            carry = do_msup(1, carry)

            @pl.when(i < nmp - 1)
            def _():
                fire_meta(1, 2 * i + 3)
            return carry

        z = jnp.int32(0)
        carry = lax.fori_loop(0, nmp, mpair, (z, z, z, z, z, z))
        pos0, out0, nf0, pos1, out1, nf1 = carry

        def finish(cc, pos, out, nf):
            st, dt, vt, _ = stgs[cc]
            wb = (nf % 2) * WSTRIDE
            zs = jnp.zeros((L,), jnp.int32)
            zt = jnp.full((L,), half, jnp.int32)
            zv = jnp.zeros((L,), jnp.float32)
            for k in range(PAIR // L):
                at = wb + pos + k * L
                st[pl.ds(at, L)] = zs
                dt[pl.ds(at, L)] = zt
                vt[pl.ds(at, L)] = zv

            @pl.when(nf > 0)
            def _():
                flush_wait(cc)
            flush_fire(cc, wb, nf)
            flush_wait(cc)
            cnt = out + ((pos + PAIR - 1) // PAIR) * PAIR
            return cnt

        cnt0 = finish(0, pos0, out0, nf0)
        cnt1 = finish(1, pos1, out1, nf1)

        for cc, cnt in ((0, cnt0), (1, cnt1)):
            cnt_stg[pl.ds(0, L)] = jnp.broadcast_to(cnt, (L,))
            pltpu.sync_copy(cnt_stg, cntP.at[pl.ds((cc * NW + w) * L, L)])

    return part


@functools.lru_cache(maxsize=None)
def _make_layer(n_nodes, e_pad):
    half = n_nodes // 2
    rcap = _rcap(e_pad)
    acc_rows = -(-(half + 1) // (NS * 8)) * (NS * 8)
    zrows_pt = acc_rows // NS
    trash = half
    out_pt = (half // NS) // 8 * 8
    out_main = out_pt * NS
    out_rem = half - out_main

    mesh = plsc.VectorSubcoreMesh(core_axis_name="c", subcore_axis_name="s")

    @functools.partial(
        pl.kernel,
        out_type=jax.ShapeDtypeStruct((n_nodes, D), jnp.float32),
        mesh=mesh,
        scratch_types=[
            pltpu.VMEM((SUP,), jnp.int32),
            pltpu.VMEM((SUP,), jnp.int32),
            pltpu.VMEM((SUP,), jnp.float32),
            pltpu.VMEM((CPS, CHUNK), jnp.int32),
            pltpu.VMEM((SUP, D), jnp.float32),
            pltpu.VMEM((SUP,), jnp.int32),
            pltpu.VMEM((SUP,), jnp.int32),
            pltpu.VMEM((SUP,), jnp.float32),
            pltpu.VMEM((CPS, CHUNK), jnp.int32),
            pltpu.VMEM((SUP, D), jnp.float32),
            pltpu.VMEM((L,), jnp.int32),
            pltpu.VMEM_SHARED(
                (-(-(n_nodes // 2 + 1) // (NS * 8)) * (NS * 8), D),
                jnp.float32),
            pltpu.SemaphoreType.DMA,
            pltpu.SemaphoreType.DMA,
            pltpu.SemaphoreType.DMA,
            pltpu.SemaphoreType.DMA,
            pltpu.SemaphoreType.DMA,
            pltpu.SemaphoreType.DMA,
        ],
        compiler_params=pltpu.CompilerParams(
            needs_layout_passes=False, use_tc_tiling_on_sc=False),
    )
    def layer(srcP, dstP, valP, cntP, table_hbm, out_hbm,
              srcA, dstA, valA, dstlocA, rowsA,
              srcB, dstB, valB, dstlocB, rowsB,
              cnt_stg, acc, msemA, msemB, gsemA, gsemB, ssemA, ssemB):
        c = lax.axis_index("c")
        s = lax.axis_index("s")
        base_node = c * half

        bufs = {
            0: (srcA, dstA, valA, dstlocA, rowsA, msemA, gsemA, ssemA),
            1: (srcB, dstB, valB, dstlocB, rowsB, msemB, gsemB, ssemB),
        }

        def fire_meta(p, rb, sup):
            src_v, dst_v, val_v = bufs[p][0], bufs[p][1], bufs[p][2]
            msem = bufs[p][5]
            mb = (rb + sup) * SUP
            pltpu.async_copy(srcP.at[pl.ds(mb, SUP)], src_v, msem)
            pltpu.async_copy(dstP.at[pl.ds(mb, SUP)], dst_v, msem)
            pltpu.async_copy(valP.at[pl.ds(mb, SUP)], val_v, msem)

        def wait_meta(p):
            src_v, dst_v, val_v = bufs[p][0], bufs[p][1], bufs[p][2]
            msem = bufs[p][5]
            pltpu.make_async_copy(srcP.at[pl.ds(0, SUP)], src_v, msem).wait()
            pltpu.make_async_copy(dstP.at[pl.ds(0, SUP)], dst_v, msem).wait()
            pltpu.make_async_copy(valP.at[pl.ds(0, SUP)], val_v, msem).wait()

        def fire_gathers(p):
            src_v, rows_v, gsem = bufs[p][0], bufs[p][4], bufs[p][6]
            for k in range(CPS):
                pltpu.async_copy(
                    table_hbm.at[src_v.at[pl.ds(k * CHUNK, CHUNK)]],
                    rows_v.at[pl.ds(k * CHUNK, CHUNK)], gsem)

        def wait_gathers(p):
            rows_v, gsem = bufs[p][4], bufs[p][6]
            pltpu.make_async_copy(table_hbm.at[pl.ds(0, SUP)], rows_v,
                                  gsem).wait()

        def fire_scatters(p):
            dstloc_v, rows_v, ssem = bufs[p][3], bufs[p][4], bufs[p][7]
            for k in range(CPS):
                pltpu.async_copy(
                    rows_v.at[pl.ds(k * CHUNK, CHUNK)],
                    acc.at[dstloc_v.at[k]], ssem, add=True)

        def wait_scatters(p):
            rows_v, ssem = bufs[p][4], bufs[p][7]
            pltpu.make_async_copy(table_hbm.at[pl.ds(0, SUP)], rows_v,
                                  ssem).wait()

        def compute(p):
            dst_v, val_v, dstloc_v, rows_v = (
                bufs[p][1], bufs[p][2], bufs[p][3], bufs[p][4])

            def blk(b, carry):
                off = b * L
                dstloc_v[b // (CHUNK // L),
                         pl.ds((b % (CHUNK // L)) * L, L)] = (
                    dst_v[pl.ds(off, L)])
                ev = jnp.broadcast_to(off, (L,))
                for j in range(L):
                    v = plsc.load_gather(val_v, [ev])
                    ev = ev + 1
                    e = off + j
                    xs = [rows_v[e, pl.ds(g * L, L)] for g in range(D // L)]
                    ys = [x * v for x in xs]
                    for g in range(D // L):
                        rows_v[e, pl.ds(g * L, L)] = ys[g]
                return carry
            lax.fori_loop(0, SUP // L, blk, 0)

        def _zr(i, carry):
            for g in range(D // L):
                rowsA[i, pl.ds(g * L, L)] = jnp.zeros((L,), jnp.float32)
            return carry
        lax.fori_loop(0, SUP, _zr, 0)

        zbase = s * zrows_pt
        nfull = zrows_pt // SUP
        zrem = zrows_pt - nfull * SUP
        for k in range(nfull):
            pltpu.sync_copy(rowsA, acc.at[pl.ds(zbase + k * SUP, SUP)])
        if zrem:
            pltpu.sync_copy(rowsA.at[pl.ds(0, zrem)],
                            acc.at[pl.ds(zbase + nfull * SUP, zrem)])
        plsc.subcore_barrier()

        def do_region(r):
            rb = (c * NW + r) * (rcap // SUP)
            pltpu.sync_copy(cntP.at[pl.ds((c * NW + r) * L, L)], cnt_stg)
            cnt = jnp.max(cnt_stg[pl.ds(0, L)])
            npairs = cnt // PAIR

            @pl.when(npairs > 0)
            def _():
                fire_meta(0, rb, 0)
                wait_meta(0)
                fire_gathers(0)
                fire_meta(1, rb, 1)

                def pair(i, carry):
                    interior = i < npairs - 1
                    wait_meta(1)

                    @pl.when(i > 0)
                    def _():
                        wait_scatters(1)
                    fire_gathers(1)
                    wait_gathers(0)
                    compute(0)
                    fire_scatters(0)

                    @pl.when(interior)
                    def _():
                        fire_meta(0, rb, 2 * i + 2)
                    wait_gathers(1)
                    compute(1)
                    fire_scatters(1)

                    @pl.when(interior)
                    def _():
                        wait_meta(0)
                        wait_scatters(0)
                        fire_gathers(0)
                        fire_meta(1, rb, 2 * i + 3)
                    return carry
                lax.fori_loop(0, npairs, pair, 0)
                wait_scatters(0)
                wait_scatters(1)

        do_region(2 * s)
        do_region(2 * s + 1)

        plsc.subcore_barrier()

        ob = s * out_pt
        pltpu.sync_copy(acc.at[pl.ds(ob, out_pt)],
                        out_hbm.at[pl.ds(base_node + ob, out_pt)])
        if out_rem:
            @pl.when(s == 0)
            def _rem():
                pltpu.sync_copy(
                    acc.at[pl.ds(out_main, out_rem)],
                    out_hbm.at[pl.ds(base_node + out_main, out_rem)])

    return layer


@functools.lru_cache(maxsize=None)
def _make_mean(n_nodes):
    blk = 1000
    grid = n_nodes // blk

    def body(a, b, c, d, o):
        o[...] = (a[...] + b[...] + c[...] + d[...]) * 0.25

    return pl.pallas_call(
        body,
        grid=(grid,),
        in_specs=[pl.BlockSpec((blk, D), lambda i: (i, 0))] * 4,
        out_specs=pl.BlockSpec((blk, D), lambda i: (i, 0)),
        out_shape=jax.ShapeDtypeStruct((n_nodes, D), jnp.float32),
    )


@jax.jit
def _impl(edge_index, edge_values, emb):
    n_nodes = emb.shape[0]
    n_edges = edge_values.shape[0]
    grp = NW * MS * 2
    e_pad = -(-n_edges // grp) * grp
    pad = e_pad - n_edges
    row = edge_index[0].astype(jnp.int32)
    col = edge_index[1].astype(jnp.int32)
    val = edge_values
    if pad:
        zi = jnp.zeros((pad,), jnp.int32)
        row = jnp.concatenate([row, zi])
        col = jnp.concatenate([col, zi])
        val = jnp.concatenate([val, jnp.zeros((pad,), val.dtype)])

    srcP, dstP, valP, cntP = _make_partition(n_nodes, e_pad)(row, col, val)
    layer = _make_layer(n_nodes, e_pad)
    embs = [emb]
    for _ in range(N_LAYERS):
        embs.append(layer(srcP, dstP, valP, cntP, embs[-1]))
    mean = _make_mean(n_nodes)(*embs)
    half = n_nodes // 2
    return mean[:half], mean[half:]


def kernel(edge_index, edge_values, emb):
    return _impl(edge_index, edge_values, emb)


# speedup vs baseline: 7.1019x; 1.0863x over previous
import functools

import jax
import jax.numpy as jnp
from jax import lax
from jax.experimental import pallas as pl
from jax.experimental.pallas import tpu as pltpu
from jax.experimental.pallas import tpu_sc as plsc

D = 64
NC = 2
NS = 16
NW = NC * NS
L = 16
CHUNK = 96
CPS = 2
SUP = CHUNK * CPS
PAIR = 2 * SUP
N_LAYERS = 3

MS = 256
W = 2304
WSTRIDE = W + PAIR + 32
SCAP = WSTRIDE + W + PAIR + 32


def _rcap(e_pad):
    span32 = e_pad // NW
    return -(-span32 // W) * W


@functools.lru_cache(maxsize=None)
def _make_partition(n_nodes, e_pad):
    half = n_nodes // 2
    span32 = e_pad // NW
    nms = span32 // MS
    nmp = nms // 2
    rcap = _rcap(e_pad)

    mesh = plsc.VectorSubcoreMesh(core_axis_name="c", subcore_axis_name="s")

    @functools.partial(
        pl.kernel,
        out_type=(
            jax.ShapeDtypeStruct((NC * NW * rcap,), jnp.int32),
            jax.ShapeDtypeStruct((NC * NW * rcap,), jnp.int32),
            jax.ShapeDtypeStruct((NC * NW * rcap,), jnp.float32),
            jax.ShapeDtypeStruct((NC * NW * L,), jnp.int32),
        ),
        mesh=mesh,
        scratch_types=[
            pltpu.VMEM((MS,), jnp.int32),
            pltpu.VMEM((MS,), jnp.int32),
            pltpu.VMEM((MS,), jnp.float32),
            pltpu.VMEM((MS,), jnp.int32),
            pltpu.VMEM((MS,), jnp.int32),
            pltpu.VMEM((MS,), jnp.float32),
            pltpu.VMEM((SCAP,), jnp.int32),
            pltpu.VMEM((SCAP,), jnp.int32),
            pltpu.VMEM((SCAP,), jnp.float32),
            pltpu.VMEM((SCAP,), jnp.int32),
            pltpu.VMEM((SCAP,), jnp.int32),
            pltpu.VMEM((SCAP,), jnp.float32),
            pltpu.VMEM((L,), jnp.int32),
            pltpu.SemaphoreType.DMA,
            pltpu.SemaphoreType.DMA,
            pltpu.SemaphoreType.DMA,
            pltpu.SemaphoreType.DMA,
        ],
        compiler_params=pltpu.CompilerParams(
            needs_layout_passes=False, use_tc_tiling_on_sc=False),
    )
    def part(dst_hbm, src_hbm, val_hbm,
             srcP, dstP, valP, cntP,
             srcMA, dstMA, valMA, srcMB, dstMB, valMB,
             stg_s0, stg_d0, stg_v0, stg_s1, stg_d1, stg_v1,
             cnt_stg, msemA, msemB, fsem0, fsem1):
        c = lax.axis_index("c")
        s = lax.axis_index("s")
        w = c * NS + s
        wbase = w * span32

        metas = {0: (srcMA, dstMA, valMA, msemA),
                 1: (srcMB, dstMB, valMB, msemB)}
        stgs = {0: (stg_s0, stg_d0, stg_v0, fsem0),
                1: (stg_s1, stg_d1, stg_v1, fsem1)}

        def fire_meta(p, m):
            sm, dm, vm, msem = metas[p]
            mb = wbase + m * MS
            pltpu.async_copy(src_hbm.at[pl.ds(mb, MS)], sm, msem)
            pltpu.async_copy(dst_hbm.at[pl.ds(mb, MS)], dm, msem)
            pltpu.async_copy(val_hbm.at[pl.ds(mb, MS)], vm, msem)

        def wait_meta(p):
            sm, dm, vm, msem = metas[p]
            pltpu.make_async_copy(src_hbm.at[pl.ds(0, MS)], sm, msem).wait()
            pltpu.make_async_copy(dst_hbm.at[pl.ds(0, MS)], dm, msem).wait()
            pltpu.make_async_copy(val_hbm.at[pl.ds(0, MS)], vm, msem).wait()

        iota = lax.iota(jnp.int32, L)

        def flush_wait(cc):
            st, dt, vt, fsem = stgs[cc]
            pltpu.make_async_copy(src_hbm.at[pl.ds(0, W)], st.at[pl.ds(0, W)],
                                  fsem).wait()
            pltpu.make_async_copy(src_hbm.at[pl.ds(0, W)], dt.at[pl.ds(0, W)],
                                  fsem).wait()
            pltpu.make_async_copy(val_hbm.at[pl.ds(0, W)], vt.at[pl.ds(0, W)],
                                  fsem).wait()

        def flush_fire(cc, wb, nf):
            st, dt, vt, fsem = stgs[cc]
            ob = ((cc * NW + w) * (rcap // W) + nf) * W
            pltpu.async_copy(st.at[pl.ds(wb, W)], srcP.at[pl.ds(ob, W)], fsem)
            pltpu.async_copy(dt.at[pl.ds(wb, W)], dstP.at[pl.ds(ob, W)], fsem)
            pltpu.async_copy(vt.at[pl.ds(wb, W)], valP.at[pl.ds(ob, W)], fsem)

        def append(cc, sv, dv, vv, mask, pos, out, nf):
            st, dt, vt, _ = stgs[cc]
            wb = (nf % 2) * WSTRIDE
            at = wb + pos
            plsc.store_compressed(st.at[pl.ds(at, L)], sv, mask=mask)
            plsc.store_compressed(dt.at[pl.ds(at, L)], dv, mask=mask)
            plsc.store_compressed(vt.at[pl.ds(at, L)], vv, mask=mask)
            n = jnp.max(plsc.all_reduce_population_count(mask))
            pos = pos + n
            cross = pos >= W

            def do_flush():
                sp_s = st[pl.ds(wb + W, L)]
                sp_d = dt[pl.ds(wb + W, L)]
                sp_v = vt[pl.ds(wb + W, L)]

                @pl.when(nf > 0)
                def _():
                    flush_wait(cc)
                flush_fire(cc, wb, nf)
                nwb = ((nf + 1) % 2) * WSTRIDE
                rem = pos - W
                m = iota < rem
                plsc.store_compressed(st.at[pl.ds(nwb, L)], sp_s, mask=m)
                plsc.store_compressed(dt.at[pl.ds(nwb, L)], sp_d, mask=m)
                plsc.store_compressed(vt.at[pl.ds(nwb, L)], sp_v, mask=m)

            @pl.when(cross)
            def _():
                do_flush()
            pos = jnp.where(cross, pos - W, pos)
            out = jnp.where(cross, out + W, out)
            nf = jnp.where(cross, nf + 1, nf)
            return pos, out, nf

        def do_msup(p, carry):
            sm, dm, vm, _ = metas[p]

            def grp(g, cr):
                pos0, out0, nf0, pos1, out1, nf1 = cr
                off = g * L
                sv = sm[pl.ds(off, L)]
                dv = dm[pl.ds(off, L)]
                vv = vm[pl.ds(off, L)]
                own0 = dv < half
                dloc = jnp.where(own0, dv, dv - half)
                pos0, out0, nf0 = append(0, sv, dloc, vv, own0,
                                         pos0, out0, nf0)
                pos1, out1, nf1 = append(1, sv, dloc, vv,
                                         jnp.logical_not(own0),
                                         pos1, out1, nf1)
                return (pos0, out0, nf0, pos1, out1, nf1)
            return lax.fori_loop(0, MS // L, grp, carry)

        fire_meta(0, 0)
        fire_meta(1, 1)

        def mpair(i, carry):
            wait_meta(0)
            carry = do_msup(0, carry)

            @pl.when(i < nmp - 1)
            def _():
                fire_meta(0, 2 * i + 2)
            wait_meta(1)
            carry = do_msup(1, carry)

            @pl.when(i < nmp - 1)
            def _():
                fire_meta(1, 2 * i + 3)
            return carry

        z = jnp.int32(0)
        carry = lax.fori_loop(0, nmp, mpair, (z, z, z, z, z, z))
        pos0, out0, nf0, pos1, out1, nf1 = carry

        def finish(cc, pos, out, nf):
            st, dt, vt, _ = stgs[cc]
            wb = (nf % 2) * WSTRIDE
            zs = jnp.zeros((L,), jnp.int32)
            zt = jnp.full((L,), half, jnp.int32)
            zv = jnp.zeros((L,), jnp.float32)
            for k in range(PAIR // L):
                at = wb + pos + k * L
                st[pl.ds(at, L)] = zs
                dt[pl.ds(at, L)] = zt
                vt[pl.ds(at, L)] = zv

            @pl.when(nf > 0)
            def _():
                flush_wait(cc)
            flush_fire(cc, wb, nf)
            flush_wait(cc)
            cnt = out + ((pos + PAIR - 1) // PAIR) * PAIR
            return cnt

        cnt0 = finish(0, pos0, out0, nf0)
        cnt1 = finish(1, pos1, out1, nf1)

        for cc, cnt in ((0, cnt0), (1, cnt1)):
            cnt_stg[pl.ds(0, L)] = jnp.broadcast_to(cnt, (L,))
            pltpu.sync_copy(cnt_stg, cntP.at[pl.ds((cc * NW + w) * L, L)])

    return part


@functools.lru_cache(maxsize=None)
def _make_layer(n_nodes, e_pad):
    half = n_nodes // 2
    rcap = _rcap(e_pad)
    acc_rows = -(-(half + 1) // (NS * 8)) * (NS * 8)
    zrows_pt = acc_rows // NS
    trash = half
    out_pt = (half // NS) // 8 * 8
    out_main = out_pt * NS
    out_rem = half - out_main

    mesh = plsc.VectorSubcoreMesh(core_axis_name="c", subcore_axis_name="s")

    @functools.partial(
        pl.kernel,
        out_type=jax.ShapeDtypeStruct((n_nodes, D), jnp.float32),
        mesh=mesh,
        scratch_types=[
            pltpu.VMEM((SUP,), jnp.int32),
            pltpu.VMEM((SUP,), jnp.int32),
            pltpu.VMEM((SUP,), jnp.float32),
            pltpu.VMEM((CPS, CHUNK), jnp.int32),
            pltpu.VMEM((SUP, D), jnp.float32),
            pltpu.VMEM((SUP,), jnp.int32),
            pltpu.VMEM((SUP,), jnp.int32),
            pltpu.VMEM((SUP,), jnp.float32),
            pltpu.VMEM((CPS, CHUNK), jnp.int32),
            pltpu.VMEM((SUP, D), jnp.float32),
            pltpu.VMEM((L,), jnp.int32),
            pltpu.VMEM_SHARED(
                (-(-(n_nodes // 2 + 1) // (NS * 8)) * (NS * 8), D),
                jnp.float32),
            pltpu.SemaphoreType.DMA,
            pltpu.SemaphoreType.DMA,
            pltpu.SemaphoreType.DMA,
            pltpu.SemaphoreType.DMA,
            pltpu.SemaphoreType.DMA,
            pltpu.SemaphoreType.DMA,
        ],
        compiler_params=pltpu.CompilerParams(
            needs_layout_passes=False, use_tc_tiling_on_sc=False),
    )
    def layer(srcP, dstP, valP, cntP, table_hbm, out_hbm,
              srcA, dstA, valA, dstlocA, rowsA,
              srcB, dstB, valB, dstlocB, rowsB,
              cnt_stg, acc, msemA, msemB, gsemA, gsemB, ssemA, ssemB):
        c = lax.axis_index("c")
        s = lax.axis_index("s")
        base_node = c * half

        bufs = {
            0: (srcA, dstA, valA, dstlocA, rowsA, msemA, gsemA, ssemA),
            1: (srcB, dstB, valB, dstlocB, rowsB, msemB, gsemB, ssemB),
        }

        def fire_meta(p, rb, sup):
            src_v, dst_v, val_v = bufs[p][0], bufs[p][1], bufs[p][2]
            msem = bufs[p][5]
            mb = (rb + sup) * SUP
            pltpu.async_copy(srcP.at[pl.ds(mb, SUP)], src_v, msem)
            pltpu.async_copy(dstP.at[pl.ds(mb, SUP)], dst_v, msem)
            pltpu.async_copy(valP.at[pl.ds(mb, SUP)], val_v, msem)

        def wait_meta(p):
            src_v, dst_v, val_v = bufs[p][0], bufs[p][1], bufs[p][2]
            msem = bufs[p][5]
            pltpu.make_async_copy(srcP.at[pl.ds(0, SUP)], src_v, msem).wait()
            pltpu.make_async_copy(dstP.at[pl.ds(0, SUP)], dst_v, msem).wait()
            pltpu.make_async_copy(valP.at[pl.ds(0, SUP)], val_v, msem).wait()

        def fire_gathers(p):
            src_v, rows_v, gsem = bufs[p][0], bufs[p][4], bufs[p][6]
            for k in range(CPS):
                pltpu.async_copy(
                    table_hbm.at[src_v.at[pl.ds(k * CHUNK, CHUNK)]],
                    rows_v.at[pl.ds(k * CHUNK, CHUNK)], gsem)

        def wait_gathers(p):
            rows_v, gsem = bufs[p][4], bufs[p][6]
            pltpu.make_async_copy(table_hbm.at[pl.ds(0, SUP)], rows_v,
                                  gsem).wait()

        def fire_scatters(p):
            dstloc_v, rows_v, ssem = bufs[p][3], bufs[p][4], bufs[p][7]
            for k in range(CPS):
                pltpu.async_copy(
                    rows_v.at[pl.ds(k * CHUNK, CHUNK)],
                    acc.at[dstloc_v.at[k]], ssem, add=True)

        def wait_scatters(p):
            rows_v, ssem = bufs[p][4], bufs[p][7]
            pltpu.make_async_copy(table_hbm.at[pl.ds(0, SUP)], rows_v,
                                  ssem).wait()

        def compute(p):
            dst_v, val_v, dstloc_v, rows_v = (
                bufs[p][1], bufs[p][2], bufs[p][3], bufs[p][4])

            def blk(b, carry):
                off = b * L
                dstloc_v[b // (CHUNK // L),
                         pl.ds((b % (CHUNK // L)) * L, L)] = (
                    dst_v[pl.ds(off, L)])
                ev = jnp.broadcast_to(off, (L,))
                for j in range(0, L, 2):
                    v0 = plsc.load_gather(val_v, [ev])
                    v1 = plsc.load_gather(val_v, [ev + 1])
                    ev = ev + 2
                    e0 = off + j
                    e1 = off + j + 1
                    xs0 = [rows_v[e0, pl.ds(g * L, L)]
                           for g in range(D // L)]
                    xs1 = [rows_v[e1, pl.ds(g * L, L)]
                           for g in range(D // L)]
                    for g in range(D // L):
                        rows_v[e0, pl.ds(g * L, L)] = xs0[g] * v0
                    for g in range(D // L):
                        rows_v[e1, pl.ds(g * L, L)] = xs1[g] * v1
                return carry
            lax.fori_loop(0, SUP // L, blk, 0)

        def _zr(i, carry):
            for g in range(D // L):
                rowsA[i, pl.ds(g * L, L)] = jnp.zeros((L,), jnp.float32)
            return carry
        lax.fori_loop(0, SUP, _zr, 0)

        zbase = s * zrows_pt
        nfull = zrows_pt // SUP
        zrem = zrows_pt - nfull * SUP
        for k in range(nfull):
            pltpu.sync_copy(rowsA, acc.at[pl.ds(zbase + k * SUP, SUP)])
        if zrem:
            pltpu.sync_copy(rowsA.at[pl.ds(0, zrem)],
                            acc.at[pl.ds(zbase + nfull * SUP, zrem)])
        plsc.subcore_barrier()

        def do_region(r):
            rb = (c * NW + r) * (rcap // SUP)
            pltpu.sync_copy(cntP.at[pl.ds((c * NW + r) * L, L)], cnt_stg)
            cnt = jnp.max(cnt_stg[pl.ds(0, L)])
            npairs = cnt // PAIR

            @pl.when(npairs > 0)
            def _():
                fire_meta(0, rb, 0)
                wait_meta(0)
                fire_gathers(0)
                fire_meta(1, rb, 1)

                def pair(i, carry):
                    interior = i < npairs - 1
                    wait_meta(1)

                    @pl.when(i > 0)
                    def _():
                        wait_scatters(1)
                    fire_gathers(1)
                    wait_gathers(0)
                    compute(0)
                    fire_scatters(0)

                    @pl.when(interior)
                    def _():
                        fire_meta(0, rb, 2 * i + 2)
                    wait_gathers(1)
                    compute(1)
                    fire_scatters(1)

                    @pl.when(interior)
                    def _():
                        wait_meta(0)
                        wait_scatters(0)
                        fire_gathers(0)
                        fire_meta(1, rb, 2 * i + 3)
                    return carry
                lax.fori_loop(0, npairs, pair, 0)
                wait_scatters(0)
                wait_scatters(1)

        do_region(2 * s)
        do_region(2 * s + 1)

        plsc.subcore_barrier()

        ob = s * out_pt
        pltpu.sync_copy(acc.at[pl.ds(ob, out_pt)],
                        out_hbm.at[pl.ds(base_node + ob, out_pt)])
        if out_rem:
            @pl.when(s == 0)
            def _rem():
                pltpu.sync_copy(
                    acc.at[pl.ds(out_main, out_rem)],
                    out_hbm.at[pl.ds(base_node + out_main, out_rem)])

    return layer


@functools.lru_cache(maxsize=None)
def _make_mean(n_nodes):
    blk = 1000
    grid = n_nodes // blk

    def body(a, b, c, d, o):
        o[...] = (a[...] + b[...] + c[...] + d[...]) * 0.25

    return pl.pallas_call(
        body,
        grid=(grid,),
        in_specs=[pl.BlockSpec((blk, D), lambda i: (i, 0))] * 4,
        out_specs=pl.BlockSpec((blk, D), lambda i: (i, 0)),
        out_shape=jax.ShapeDtypeStruct((n_nodes, D), jnp.float32),
    )


@jax.jit
def _impl(edge_index, edge_values, emb):
    n_nodes = emb.shape[0]
    n_edges = edge_values.shape[0]
    grp = NW * MS * 2
    e_pad = -(-n_edges // grp) * grp
    pad = e_pad - n_edges
    row = edge_index[0].astype(jnp.int32)
    col = edge_index[1].astype(jnp.int32)
    val = edge_values
    if pad:
        zi = jnp.zeros((pad,), jnp.int32)
        row = jnp.concatenate([row, zi])
        col = jnp.concatenate([col, zi])
        val = jnp.concatenate([val, jnp.zeros((pad,), val.dtype)])

    srcP, dstP, valP, cntP = _make_partition(n_nodes, e_pad)(row, col, val)
    layer = _make_layer(n_nodes, e_pad)
    embs = [emb]
    for _ in range(N_LAYERS):
        embs.append(layer(srcP, dstP, valP, cntP, embs[-1]))
    mean = _make_mean(n_nodes)(*embs)
    half = n_nodes // 2
    return mean[:half], mean[half:]


def kernel(edge_index, edge_values, emb):
    return _impl(edge_index, edge_values, emb)


# speedup vs baseline: 7.1162x; 1.0020x over previous
import functools

import jax
import jax.numpy as jnp
from jax import lax
from jax.experimental import pallas as pl
from jax.experimental.pallas import tpu as pltpu
from jax.experimental.pallas import tpu_sc as plsc

D = 64
NC = 2
NS = 16
NW = NC * NS
L = 16
CHUNK = 64
CPS = 3
SUP = CHUNK * CPS
PAIR = 2 * SUP
N_LAYERS = 3

MS = 256
W = 2304
WSTRIDE = W + PAIR + 32
SCAP = WSTRIDE + W + PAIR + 32


def _rcap(e_pad):
    span32 = e_pad // NW
    return -(-span32 // W) * W


@functools.lru_cache(maxsize=None)
def _make_partition(n_nodes, e_pad):
    half = n_nodes // 2
    span32 = e_pad // NW
    nms = span32 // MS
    nmp = nms // 2
    rcap = _rcap(e_pad)

    mesh = plsc.VectorSubcoreMesh(core_axis_name="c", subcore_axis_name="s")

    @functools.partial(
        pl.kernel,
        out_type=(
            jax.ShapeDtypeStruct((NC * NW * rcap,), jnp.int32),
            jax.ShapeDtypeStruct((NC * NW * rcap,), jnp.int32),
            jax.ShapeDtypeStruct((NC * NW * rcap,), jnp.float32),
            jax.ShapeDtypeStruct((NC * NW * L,), jnp.int32),
        ),
        mesh=mesh,
        scratch_types=[
            pltpu.VMEM((MS,), jnp.int32),
            pltpu.VMEM((MS,), jnp.int32),
            pltpu.VMEM((MS,), jnp.float32),
            pltpu.VMEM((MS,), jnp.int32),
            pltpu.VMEM((MS,), jnp.int32),
            pltpu.VMEM((MS,), jnp.float32),
            pltpu.VMEM((SCAP,), jnp.int32),
            pltpu.VMEM((SCAP,), jnp.int32),
            pltpu.VMEM((SCAP,), jnp.float32),
            pltpu.VMEM((SCAP,), jnp.int32),
            pltpu.VMEM((SCAP,), jnp.int32),
            pltpu.VMEM((SCAP,), jnp.float32),
            pltpu.VMEM((L,), jnp.int32),
            pltpu.SemaphoreType.DMA,
            pltpu.SemaphoreType.DMA,
            pltpu.SemaphoreType.DMA,
            pltpu.SemaphoreType.DMA,
        ],
        compiler_params=pltpu.CompilerParams(
            needs_layout_passes=False, use_tc_tiling_on_sc=False),
    )
    def part(dst_hbm, src_hbm, val_hbm,
             srcP, dstP, valP, cntP,
             srcMA, dstMA, valMA, srcMB, dstMB, valMB,
             stg_s0, stg_d0, stg_v0, stg_s1, stg_d1, stg_v1,
             cnt_stg, msemA, msemB, fsem0, fsem1):
        c = lax.axis_index("c")
        s = lax.axis_index("s")
        w = c * NS + s
        wbase = w * span32

        metas = {0: (srcMA, dstMA, valMA, msemA),
                 1: (srcMB, dstMB, valMB, msemB)}
        stgs = {0: (stg_s0, stg_d0, stg_v0, fsem0),
                1: (stg_s1, stg_d1, stg_v1, fsem1)}

        def fire_meta(p, m):
            sm, dm, vm, msem = metas[p]
            mb = wbase + m * MS
            pltpu.async_copy(src_hbm.at[pl.ds(mb, MS)], sm, msem)
            pltpu.async_copy(dst_hbm.at[pl.ds(mb, MS)], dm, msem)
            pltpu.async_copy(val_hbm.at[pl.ds(mb, MS)], vm, msem)

        def wait_meta(p):
            sm, dm, vm, msem = metas[p]
            pltpu.make_async_copy(src_hbm.at[pl.ds(0, MS)], sm, msem).wait()
            pltpu.make_async_copy(dst_hbm.at[pl.ds(0, MS)], dm, msem).wait()
            pltpu.make_async_copy(val_hbm.at[pl.ds(0, MS)], vm, msem).wait()

        iota = lax.iota(jnp.int32, L)

        def flush_wait(cc):
            st, dt, vt, fsem = stgs[cc]
            pltpu.make_async_copy(src_hbm.at[pl.ds(0, W)], st.at[pl.ds(0, W)],
                                  fsem).wait()
            pltpu.make_async_copy(src_hbm.at[pl.ds(0, W)], dt.at[pl.ds(0, W)],
                                  fsem).wait()
            pltpu.make_async_copy(val_hbm.at[pl.ds(0, W)], vt.at[pl.ds(0, W)],
                                  fsem).wait()

        def flush_fire(cc, wb, nf):
            st, dt, vt, fsem = stgs[cc]
            ob = ((cc * NW + w) * (rcap // W) + nf) * W
            pltpu.async_copy(st.at[pl.ds(wb, W)], srcP.at[pl.ds(ob, W)], fsem)
            pltpu.async_copy(dt.at[pl.ds(wb, W)], dstP.at[pl.ds(ob, W)], fsem)
            pltpu.async_copy(vt.at[pl.ds(wb, W)], valP.at[pl.ds(ob, W)], fsem)

        def append(cc, sv, dv, vv, mask, pos, out, nf):
            st, dt, vt, _ = stgs[cc]
            wb = (nf % 2) * WSTRIDE
            at = wb + pos
            plsc.store_compressed(st.at[pl.ds(at, L)], sv, mask=mask)
            plsc.store_compressed(dt.at[pl.ds(at, L)], dv, mask=mask)
            plsc.store_compressed(vt.at[pl.ds(at, L)], vv, mask=mask)
            n = jnp.max(plsc.all_reduce_population_count(mask))
            pos = pos + n
            cross = pos >= W

            def do_flush():
                sp_s = st[pl.ds(wb + W, L)]
                sp_d = dt[pl.ds(wb + W, L)]
                sp_v = vt[pl.ds(wb + W, L)]

                @pl.when(nf > 0)
                def _():
                    flush_wait(cc)
                flush_fire(cc, wb, nf)
                nwb = ((nf + 1) % 2) * WSTRIDE
                rem = pos - W
                m = iota < rem
                plsc.store_compressed(st.at[pl.ds(nwb, L)], sp_s, mask=m)
                plsc.store_compressed(dt.at[pl.ds(nwb, L)], sp_d, mask=m)
                plsc.store_compressed(vt.at[pl.ds(nwb, L)], sp_v, mask=m)

            @pl.when(cross)
            def _():
                do_flush()
            pos = jnp.where(cross, pos - W, pos)
            out = jnp.where(cross, out + W, out)
            nf = jnp.where(cross, nf + 1, nf)
            return pos, out, nf

        def do_msup(p, carry):
            sm, dm, vm, _ = metas[p]

            def grp(g, cr):
                pos0, out0, nf0, pos1, out1, nf1 = cr
                off = g * L
                sv = sm[pl.ds(off, L)]
                dv = dm[pl.ds(off, L)]
                vv = vm[pl.ds(off, L)]
                own0 = dv < half
                dloc = jnp.where(own0, dv, dv - half)
                pos0, out0, nf0 = append(0, sv, dloc, vv, own0,
                                         pos0, out0, nf0)
                pos1, out1, nf1 = append(1, sv, dloc, vv,
                                         jnp.logical_not(own0),
                                         pos1, out1, nf1)
                return (pos0, out0, nf0, pos1, out1, nf1)
            return lax.fori_loop(0, MS // L, grp, carry)

        fire_meta(0, 0)
        fire_meta(1, 1)

        def mpair(i, carry):
            wait_meta(0)
            carry = do_msup(0, carry)

            @pl.when(i < nmp - 1)
            def _():
                fire_meta(0, 2 * i + 2)
            wait_meta(1)
            carry = do_msup(1, carry)

            @pl.when(i < nmp - 1)
            def _():
                fire_meta(1, 2 * i + 3)
            return carry

        z = jnp.int32(0)
        carry = lax.fori_loop(0, nmp, mpair, (z, z, z, z, z, z))
        pos0, out0, nf0, pos1, out1, nf1 = carry

        def finish(cc, pos, out, nf):
            st, dt, vt, _ = stgs[cc]
            wb = (nf % 2) * WSTRIDE
            zs = jnp.zeros((L,), jnp.int32)
            zt = jnp.full((L,), half, jnp.int32)
            zv = jnp.zeros((L,), jnp.float32)
            for k in range(PAIR // L):
                at = wb + pos + k * L
                st[pl.ds(at, L)] = zs
                dt[pl.ds(at, L)] = zt
                vt[pl.ds(at, L)] = zv

            @pl.when(nf > 0)
            def _():
                flush_wait(cc)
            flush_fire(cc, wb, nf)
            flush_wait(cc)
            cnt = out + ((pos + PAIR - 1) // PAIR) * PAIR
            return cnt

        cnt0 = finish(0, pos0, out0, nf0)
        cnt1 = finish(1, pos1, out1, nf1)

        for cc, cnt in ((0, cnt0), (1, cnt1)):
            cnt_stg[pl.ds(0, L)] = jnp.broadcast_to(cnt, (L,))
            pltpu.sync_copy(cnt_stg, cntP.at[pl.ds((cc * NW + w) * L, L)])

    return part


@functools.lru_cache(maxsize=None)
def _make_layer(n_nodes, e_pad):
    half = n_nodes // 2
    rcap = _rcap(e_pad)
    acc_rows = -(-(half + 1) // (NS * 8)) * (NS * 8)
    zrows_pt = acc_rows // NS
    trash = half
    out_pt = (half // NS) // 8 * 8
    out_main = out_pt * NS
    out_rem = half - out_main

    mesh = plsc.VectorSubcoreMesh(core_axis_name="c", subcore_axis_name="s")

    @functools.partial(
        pl.kernel,
        out_type=jax.ShapeDtypeStruct((n_nodes, D), jnp.float32),
        mesh=mesh,
        scratch_types=[
            pltpu.VMEM((SUP,), jnp.int32),
            pltpu.VMEM((SUP,), jnp.int32),
            pltpu.VMEM((SUP,), jnp.float32),
            pltpu.VMEM((CPS, CHUNK), jnp.int32),
            pltpu.VMEM((SUP, D), jnp.float32),
            pltpu.VMEM((SUP,), jnp.int32),
            pltpu.VMEM((SUP,), jnp.int32),
            pltpu.VMEM((SUP,), jnp.float32),
            pltpu.VMEM((CPS, CHUNK), jnp.int32),
            pltpu.VMEM((SUP, D), jnp.float32),
            pltpu.VMEM((L,), jnp.int32),
            pltpu.VMEM_SHARED(
                (-(-(n_nodes // 2 + 1) // (NS * 8)) * (NS * 8), D),
                jnp.float32),
            pltpu.SemaphoreType.DMA,
            pltpu.SemaphoreType.DMA,
            pltpu.SemaphoreType.DMA,
            pltpu.SemaphoreType.DMA,
            pltpu.SemaphoreType.DMA,
            pltpu.SemaphoreType.DMA,
        ],
        compiler_params=pltpu.CompilerParams(
            needs_layout_passes=False, use_tc_tiling_on_sc=False),
    )
    def layer(srcP, dstP, valP, cntP, table_hbm, out_hbm,
              srcA, dstA, valA, dstlocA, rowsA,
              srcB, dstB, valB, dstlocB, rowsB,
              cnt_stg, acc, msemA, msemB, gsemA, gsemB, ssemA, ssemB):
        c = lax.axis_index("c")
        s = lax.axis_index("s")
        base_node = c * half

        bufs = {
            0: (srcA, dstA, valA, dstlocA, rowsA, msemA, gsemA, ssemA),
            1: (srcB, dstB, valB, dstlocB, rowsB, msemB, gsemB, ssemB),
        }

        def fire_meta(p, rb, sup):
            src_v, dst_v, val_v = bufs[p][0], bufs[p][1], bufs[p][2]
            msem = bufs[p][5]
            mb = (rb + sup) * SUP
            pltpu.async_copy(srcP.at[pl.ds(mb, SUP)], src_v, msem)
            pltpu.async_copy(dstP.at[pl.ds(mb, SUP)], dst_v, msem)
            pltpu.async_copy(valP.at[pl.ds(mb, SUP)], val_v, msem)

        def wait_meta(p):
            src_v, dst_v, val_v = bufs[p][0], bufs[p][1], bufs[p][2]
            msem = bufs[p][5]
            pltpu.make_async_copy(srcP.at[pl.ds(0, SUP)], src_v, msem).wait()
            pltpu.make_async_copy(dstP.at[pl.ds(0, SUP)], dst_v, msem).wait()
            pltpu.make_async_copy(valP.at[pl.ds(0, SUP)], val_v, msem).wait()

        def fire_gathers(p):
            src_v, rows_v, gsem = bufs[p][0], bufs[p][4], bufs[p][6]
            for k in range(CPS):
                pltpu.async_copy(
                    table_hbm.at[src_v.at[pl.ds(k * CHUNK, CHUNK)]],
                    rows_v.at[pl.ds(k * CHUNK, CHUNK)], gsem)

        def wait_gathers(p):
            rows_v, gsem = bufs[p][4], bufs[p][6]
            pltpu.make_async_copy(table_hbm.at[pl.ds(0, SUP)], rows_v,
                                  gsem).wait()

        def fire_scatters(p):
            dstloc_v, rows_v, ssem = bufs[p][3], bufs[p][4], bufs[p][7]
            for k in range(CPS):
                pltpu.async_copy(
                    rows_v.at[pl.ds(k * CHUNK, CHUNK)],
                    acc.at[dstloc_v.at[k]], ssem, add=True)

        def wait_scatters(p):
            rows_v, ssem = bufs[p][4], bufs[p][7]
            pltpu.make_async_copy(table_hbm.at[pl.ds(0, SUP)], rows_v,
                                  ssem).wait()

        def compute(p):
            dst_v, val_v, dstloc_v, rows_v = (
                bufs[p][1], bufs[p][2], bufs[p][3], bufs[p][4])

            def blk(b, carry):
                off = b * L
                dstloc_v[b // (CHUNK // L),
                         pl.ds((b % (CHUNK // L)) * L, L)] = (
                    dst_v[pl.ds(off, L)])
                ev = jnp.broadcast_to(off, (L,))
                for j in range(0, L, 2):
                    v0 = plsc.load_gather(val_v, [ev])
                    v1 = plsc.load_gather(val_v, [ev + 1])
                    ev = ev + 2
                    e0 = off + j
                    e1 = off + j + 1
                    xs0 = [rows_v[e0, pl.ds(g * L, L)]
                           for g in range(D // L)]
                    xs1 = [rows_v[e1, pl.ds(g * L, L)]
                           for g in range(D // L)]
                    for g in range(D // L):
                        rows_v[e0, pl.ds(g * L, L)] = xs0[g] * v0
                    for g in range(D // L):
                        rows_v[e1, pl.ds(g * L, L)] = xs1[g] * v1
                return carry
            lax.fori_loop(0, SUP // L, blk, 0)

        def _zr(i, carry):
            for g in range(D // L):
                rowsA[i, pl.ds(g * L, L)] = jnp.zeros((L,), jnp.float32)
            return carry
        lax.fori_loop(0, SUP, _zr, 0)

        zbase = s * zrows_pt
        nfull = zrows_pt // SUP
        zrem = zrows_pt - nfull * SUP
        for k in range(nfull):
            pltpu.sync_copy(rowsA, acc.at[pl.ds(zbase + k * SUP, SUP)])
        if zrem:
            pltpu.sync_copy(rowsA.at[pl.ds(0, zrem)],
                            acc.at[pl.ds(zbase + nfull * SUP, zrem)])
        plsc.subcore_barrier()

        def do_region(r):
            rb = (c * NW + r) * (rcap // SUP)
            pltpu.sync_copy(cntP.at[pl.ds((c * NW + r) * L, L)], cnt_stg)
            cnt = jnp.max(cnt_stg[pl.ds(0, L)])
            npairs = cnt // PAIR

            @pl.when(npairs > 0)
            def _():
                fire_meta(0, rb, 0)
                wait_meta(0)
                fire_gathers(0)
                fire_meta(1, rb, 1)

                def pair(i, carry):
                    interior = i < npairs - 1
                    wait_meta(1)

                    @pl.when(i > 0)
                    def _():
                        wait_scatters(1)
                    fire_gathers(1)
                    wait_gathers(0)
                    compute(0)
                    fire_scatters(0)

                    @pl.when(interior)
                    def _():
                        fire_meta(0, rb, 2 * i + 2)
                    wait_gathers(1)
                    compute(1)
                    fire_scatters(1)

                    @pl.when(interior)
                    def _():
                        wait_meta(0)
                        wait_scatters(0)
                        fire_gathers(0)
                        fire_meta(1, rb, 2 * i + 3)
                    return carry
                lax.fori_loop(0, npairs, pair, 0)
                wait_scatters(0)
                wait_scatters(1)

        do_region(2 * s)
        do_region(2 * s + 1)

        plsc.subcore_barrier()

        ob = s * out_pt
        pltpu.sync_copy(acc.at[pl.ds(ob, out_pt)],
                        out_hbm.at[pl.ds(base_node + ob, out_pt)])
        if out_rem:
            @pl.when(s == 0)
            def _rem():
                pltpu.sync_copy(
                    acc.at[pl.ds(out_main, out_rem)],
                    out_hbm.at[pl.ds(base_node + out_main, out_rem)])

    return layer


@functools.lru_cache(maxsize=None)
def _make_mean(n_nodes):
    blk = 1000
    grid = n_nodes // blk

    def body(a, b, c, d, o):
        o[...] = (a[...] + b[...] + c[...] + d[...]) * 0.25

    return pl.pallas_call(
        body,
        grid=(grid,),
        in_specs=[pl.BlockSpec((blk, D), lambda i: (i, 0))] * 4,
        out_specs=pl.BlockSpec((blk, D), lambda i: (i, 0)),
        out_shape=jax.ShapeDtypeStruct((n_nodes, D), jnp.float32),
    )


@jax.jit
def _impl(edge_index, edge_values, emb):
    n_nodes = emb.shape[0]
    n_edges = edge_values.shape[0]
    grp = NW * MS * 2
    e_pad = -(-n_edges // grp) * grp
    pad = e_pad - n_edges
    row = edge_index[0].astype(jnp.int32)
    col = edge_index[1].astype(jnp.int32)
    val = edge_values
    if pad:
        zi = jnp.zeros((pad,), jnp.int32)
        row = jnp.concatenate([row, zi])
        col = jnp.concatenate([col, zi])
        val = jnp.concatenate([val, jnp.zeros((pad,), val.dtype)])

    srcP, dstP, valP, cntP = _make_partition(n_nodes, e_pad)(row, col, val)
    layer = _make_layer(n_nodes, e_pad)
    embs = [emb]
    for _ in range(N_LAYERS):
        embs.append(layer(srcP, dstP, valP, cntP, embs[-1]))
    mean = _make_mean(n_nodes)(*embs)
    half = n_nodes // 2
    return mean[:half], mean[half:]


def kernel(edge_index, edge_values, emb):
    return _impl(edge_index, edge_values, emb)
